# Initial kernel scaffold; baseline (speedup 1.0000x reference)
import jax
import jax.numpy as jnp
from jax.experimental import pallas as pl


def kernel(recon, target, d):
    raise NotImplementedError("write your pallas kernel here")


# speedup vs baseline: 8.6684x; 8.6684x over previous
import functools

import jax
import jax.numpy as jnp
from jax.experimental import pallas as pl
from jax.experimental.pallas import tpu as pltpu

_BATCH = 4
_N = 8192
_NA = 64
_AS = 128
_NSTEPS = 8
_NSTREAMS = 2 * _BATCH
_NBLK = _N // _AS
_BIG_I = 2 ** 30


def _clamped_log(x):
    safe = jnp.where(x > 0, x, 1.0)
    return jnp.maximum(jnp.where(x > 0, jnp.log(safe), -100.0), -100.0)


def _mp_kernel(x_ref, m12_ref, d_ref, out_ref, f_ref, rv_ref, ri_ref, rit_ref):
    s = pl.program_id(0)
    m12 = m12_ref[...]

    d = d_ref[...]
    nsq_col = jnp.sum(d * d, axis=1, keepdims=True)
    eye = (jax.lax.broadcasted_iota(jnp.int32, (_NA, _NA), 0)
           == jax.lax.broadcasted_iota(jnp.int32, (_NA, _NA), 1)).astype(jnp.float32)
    nsq_row = jax.lax.dot_general(
        jnp.ones((1, _NA), jnp.float32), nsq_col * eye,
        (((1,), (0,)), ((), ())),
        precision=jax.lax.Precision.HIGHEST,
        preferred_element_type=jnp.float32)
    inv_row = 1.0 / (jnp.sqrt(nsq_row) + 1e-12)

    cols = jax.lax.broadcasted_iota(jnp.int32, (_NA, _NA * _AS), 1)
    onehot = ((cols & (_NA - 1))
              == jax.lax.broadcasted_iota(jnp.int32, (_NA, _NA * _AS), 0)
              ).astype(jnp.float32)
    scale = jax.lax.dot_general(
        inv_row, onehot, (((1,), (0,)), ((), ())),
        precision=jax.lax.Precision.HIGHEST,
        preferred_element_type=jnp.float32)

    f_ref[...] = jax.lax.dot_general(
        x_ref[...], m12, (((1,), (0,)), ((), ())),
        precision=jax.lax.Precision.HIGHEST,
        preferred_element_type=jnp.float32) * scale

    ii = jax.lax.broadcasted_iota(jnp.int32, (_NBLK, _NA * _AS), 0)
    cc = jax.lax.broadcasted_iota(jnp.int32, (_NBLK, _NA * _AS), 1)
    ref_idx = ((cc & (_NA - 1)) << 13) + (ii << 7) + (cc >> 6)

    lane128 = jax.lax.broadcasted_iota(jnp.int32, (1, _AS), 1)
    lane8 = jax.lax.broadcasted_iota(jnp.int32, (1, _NSTEPS), 1)
    zrow = jnp.zeros((1, _AS), jnp.float32)

    def step(t, carry):
        slab = f_ref[...]
        v = jnp.max(slab)
        cand = jnp.where(slab == v, ref_idx, _BIG_I)
        idx = jnp.min(cand)
        ai = idx >> 13
        p = idx & (_N - 1)
        i0 = p >> 7
        o = p & (_AS - 1)

        mt = lane8 == t
        rv_ref[pl.ds(s, 1), :] = jnp.where(mt, v, rv_ref[pl.ds(s, 1), :])
        ri_ref[pl.ds(s, 1), :] = jnp.where(mt, idx, ri_ref[pl.ds(s, 1), :])
        ms = lane8 == s
        rit_ref[pl.ds(t, 1), :] = jnp.where(ms, idx, rit_ref[pl.ds(t, 1), :])

        d_row = d_ref[pl.ds(ai, 1), :]
        inv_ai = 1.0 / (jnp.sqrt(jnp.sum(d_row * d_row)) + 1e-12)
        rolled = pltpu.roll(d_row, o, 1)
        db0 = jnp.where(lane128 >= o, rolled, 0.0)
        db1 = rolled - db0
        db1 = jnp.where(i0 < _NBLK - 1, db1, 0.0)
        u = jnp.concatenate(
            [jnp.concatenate([zrow, db0], axis=1),
             jnp.concatenate([db0, db1], axis=1),
             jnp.concatenate([db1, zrow], axis=1)], axis=0)
        df3 = jax.lax.dot_general(
            u, m12, (((1,), (0,)), ((), ())),
            precision=jax.lax.Precision.HIGHEST,
            preferred_element_type=jnp.float32) * (scale * (-v * inv_ai))

        @pl.when(i0 > 0)
        def _():
            f_ref[pl.ds(i0 - 1, 1), :] += df3[0:1, :]

        f_ref[pl.ds(i0, 1), :] += df3[1:2, :]

        @pl.when(i0 < _NBLK - 1)
        def _():
            f_ref[pl.ds(i0 + 1, 1), :] += df3[2:3, :]

        return carry

    jax.lax.fori_loop(0, _NSTEPS, step, 0)

    @pl.when(s == _NSTREAMS - 1)
    def _loss():
        lt = (jax.lax.broadcasted_iota(jnp.int32, (_NSTEPS, _NSTEPS), 1)
              < jax.lax.broadcasted_iota(jnp.int32, (_NSTEPS, _NSTEPS), 0))
        accs, firsts, maccs = [], [], []
        for q in range(_NSTREAMS):
            col = rit_ref[:, q:q + 1]
            row = ri_ref[q:q + 1, :]
            vrow = rv_ref[q:q + 1, :]
            eq = col == row
            acc = jnp.sum(jnp.where(eq, vrow, 0.0), axis=1, keepdims=True)
            prior = jnp.sum(jnp.where(eq & lt, 1.0, 0.0), axis=1, keepdims=True)
            first = prior == 0.0
            accs.append(acc)
            firsts.append(first)
            maccs.append(jnp.max(jnp.where(first, acc, -jnp.inf)))
        mx = jnp.maximum(0.0, functools.reduce(jnp.maximum, maccs))

        total = 0.0
        for b in range(_BATCH):
            tb = b + _BATCH
            rcol = rit_ref[:, b:b + 1]
            t_row_i = ri_ref[tb:tb + 1, :]
            t_row_v = rv_ref[tb:tb + 1, :]
            eqc = rcol == t_row_i
            t_at_r = jnp.sum(jnp.where(eqc, t_row_v, 0.0), axis=1,
                             keepdims=True) / mx
            rn = accs[b] / mx
            ell = -(t_at_r * _clamped_log(rn)
                    + (1.0 - t_at_r) * _clamped_log(1.0 - rn))
            total += jnp.sum(jnp.where(firsts[b], ell, 0.0))
            tcol = rit_ref[:, tb:tb + 1]
            r_row_i = ri_ref[b:b + 1, :]
            in_r = jnp.sum((tcol == r_row_i).astype(jnp.float32), axis=1,
                           keepdims=True) > 0.0
            tn = accs[tb] / mx
            total += jnp.sum(jnp.where(firsts[tb] & jnp.logical_not(in_r),
                                       100.0 * tn, 0.0))

        out_ref[...] = jnp.full((8, 128), total / (_BATCH * _NA * _N),
                                jnp.float32)


def kernel(recon, target, d):
    sig = jnp.concatenate([recon.reshape(_BATCH, _N),
                           target.reshape(_BATCH, _N)], axis=0)
    blk = jnp.pad(sig, ((0, 0), (0, _AS))).reshape(_NSTREAMS, _NBLK + 1, _AS)
    x = jnp.concatenate([blk[:, :_NBLK, :], blk[:, 1:, :]],
                        axis=-1).reshape(_NSTREAMS * _NBLK, 2 * _AS)
    vall = jnp.pad(d, ((0, 0), (0, 2 * _AS - _AS + 1)))
    tiled = jnp.tile(vall, (1, _AS))[:, :_AS * 2 * _AS].reshape(_NA, _AS, 2 * _AS)
    m12 = jnp.transpose(tiled, (2, 1, 0)).reshape(2 * _AS, _NA * _AS)

    out = pl.pallas_call(
        _mp_kernel,
        grid=(_NSTREAMS,),
        in_specs=[
            pl.BlockSpec((_NBLK, 2 * _AS), lambda s: (s, 0)),
            pl.BlockSpec((2 * _AS, _NA * _AS), lambda s: (0, 0)),
            pl.BlockSpec((_NA, _AS), lambda s: (0, 0)),
        ],
        out_specs=pl.BlockSpec((8, 128), lambda s: (0, 0)),
        out_shape=jax.ShapeDtypeStruct((8, 128), jnp.float32),
        scratch_shapes=[
            pltpu.VMEM((_NBLK, _NA * _AS), jnp.float32),
            pltpu.VMEM((_NSTEPS, _NSTEPS), jnp.float32),
            pltpu.VMEM((_NSTEPS, _NSTEPS), jnp.int32),
            pltpu.VMEM((_NSTEPS, _NSTEPS), jnp.int32),
        ],
        compiler_params=pltpu.CompilerParams(
            dimension_semantics=("arbitrary",)),
    )(x, m12, d)
    return out[0, 0]


# speedup vs baseline: 20.4696x; 2.3614x over previous
import functools

import jax
import jax.numpy as jnp
from jax.experimental import pallas as pl
from jax.experimental.pallas import tpu as pltpu

_BATCH = 4
_N = 8192
_NA = 64
_AS = 128
_NSTEPS = 8
_NSTREAMS = 2 * _BATCH
_NBLK = _N // _AS
_C = _NA * _AS
_BIG_I = 2 ** 30


def _clamped_log(x):
    safe = jnp.where(x > 0, x, 1.0)
    return jnp.maximum(jnp.where(x > 0, jnp.log(safe), -100.0), -100.0)


def _dot(a, b):
    return jax.lax.dot_general(
        a, b, (((1,), (0,)), ((), ())),
        precision=jax.lax.Precision.HIGHEST,
        preferred_element_type=jnp.float32)


def _mp_kernel(x_ref, m12_ref, d_ref, out_ref,
               f_ref, gm_ref, dn_ref, rmax_ref, ridx_ref,
               rv_ref, ri_ref, rit_ref):
    m12 = m12_ref[...]
    d = d_ref[...]
    nsq = jnp.sum(d * d, axis=1, keepdims=True)
    inv_col = 1.0 / (jnp.sqrt(nsq) + 1e-12)
    dn_ref[...] = d * inv_col

    eye = (jax.lax.broadcasted_iota(jnp.int32, (_NA, _NA), 0)
           == jax.lax.broadcasted_iota(jnp.int32, (_NA, _NA), 1)).astype(jnp.float32)
    inv_row = _dot(jnp.ones((1, _NA), jnp.float32), inv_col * eye)
    cols = jax.lax.broadcasted_iota(jnp.int32, (_NA, _C), 1)
    onehot = ((cols & (_NA - 1))
              == jax.lax.broadcasted_iota(jnp.int32, (_NA, _C), 0)
              ).astype(jnp.float32)
    scale = _dot(inv_row, onehot)

    for c in range(4):
        f_ref[c * 128:(c + 1) * 128, :] = (
            _dot(x_ref[c * 128:(c + 1) * 128, :], m12) * scale)

    r2 = ((jax.lax.broadcasted_iota(jnp.int32, (2 * _NA, _NA), 0) >> 1)
          == jax.lax.broadcasted_iota(jnp.int32, (2 * _NA, _NA), 1)
          ).astype(jnp.float32)
    dup2 = _dot(r2, dn_ref[...])
    bsel = jax.lax.broadcasted_iota(jnp.int32, (2 * _NA, 1), 0) & 1
    xg = jnp.concatenate([jnp.where(bsel == 1, dup2, 0.0),
                          jnp.where(bsel == 0, dup2, 0.0)], axis=1)
    gm_ref[...] = _dot(xg, m12) * scale

    refc = (((jax.lax.broadcasted_iota(jnp.int32, (_NBLK, _C), 1) & (_NA - 1))
             << 13)
            | (jax.lax.broadcasted_iota(jnp.int32, (_NBLK, _C), 1) >> 6))
    blkc = jax.lax.broadcasted_iota(jnp.int32, (_NBLK, _C), 0) << 7
    refmat = refc | blkc
    lane8 = jax.lax.broadcasted_iota(jnp.int32, (1, _NSTEPS), 1)
    for s in range(_NSTREAMS):
        slab = f_ref[s * _NBLK:(s + 1) * _NBLK, :]
        rm = jnp.max(slab, axis=1, keepdims=True)
        ri = jnp.min(jnp.where(slab == rm, refmat, _BIG_I), axis=1,
                     keepdims=True)
        ms = lane8 == s
        rmax_ref[...] = jnp.where(ms, rm, rmax_ref[...])
        ridx_ref[...] = jnp.where(ms, ri, ridx_ref[...])

    rlane = jax.lax.broadcasted_iota(jnp.int32, (1, _C), 1) >> 6
    lane128 = jax.lax.broadcasted_iota(jnp.int32, (1, _AS), 1)
    refc1 = refc[0:1, :]

    def step(t, carry):
        mt = lane8 == t
        for s in range(_NSTREAMS):
            col = rmax_ref[:, s:s + 1]
            v = jnp.max(col)
            idx = jnp.min(jnp.where(col == v, ridx_ref[:, s:s + 1], _BIG_I))
            ai = idx >> 13
            p = idx & (_N - 1)
            i0 = p >> 7
            o = p & (_AS - 1)

            rv_ref[s:s + 1, :] = jnp.where(mt, v, rv_ref[s:s + 1, :])
            ri_ref[s:s + 1, :] = jnp.where(mt, idx, ri_ref[s:s + 1, :])
            rit_ref[pl.ds(t, 1), s:s + 1] = jnp.full((1, 1), idx, jnp.int32)

            base = s * _NBLK + i0

            @pl.when(i0 < _NBLK - 1)
            def _main():
                g_lo = gm_ref[pl.ds(2 * ai, 1), :]
                g_hi = gm_ref[pl.ds(2 * ai + 1, 1), :]
                rhi = pltpu.roll(g_hi, 64 * o, 1)
                rlo = pltpu.roll(g_lo, 64 * o, 1)
                rge = rlane >= o

                @pl.when(i0 > 0)
                def _():
                    f_ref[pl.ds(base - 1, 1), :] += (
                        jnp.where(rge, rlo, 0.0) * (-v))

                f_ref[pl.ds(base, 1), :] += jnp.where(rge, rhi, rlo) * (-v)
                f_ref[pl.ds(base + 1, 1), :] += (
                    jnp.where(rge, 0.0, rhi) * (-v))

            @pl.when(i0 == _NBLK - 1)
            def _trunc():
                dnr = dn_ref[pl.ds(ai, 1), :]
                rolled = pltpu.roll(dnr, o, 1)
                db0 = jnp.where(lane128 >= o, rolled, 0.0)
                zr = jnp.zeros((1, _AS), jnp.float32)
                u2 = jnp.concatenate(
                    [jnp.concatenate([zr, db0], axis=1),
                     jnp.concatenate([db0, zr], axis=1)], axis=0)
                df2 = _dot(u2, m12) * (scale * (-v))
                f_ref[pl.ds(base - 1, 1), :] += df2[0:1, :]
                f_ref[pl.ds(base, 1), :] += df2[1:2, :]

            i0m = jnp.clip(i0 - 1, 0, _NBLK - 3)
            ms = lane8 == s
            for j in range(3):
                rowf = f_ref[pl.ds(s * _NBLK + i0m + j, 1), :]
                m1 = jnp.max(rowf, axis=1, keepdims=True)
                r1 = jnp.min(jnp.where(rowf == m1,
                                       refc1 | ((i0m + j) << 7), _BIG_I),
                             axis=1, keepdims=True)
                rmax_ref[pl.ds(i0m + j, 1), :] = jnp.where(
                    ms, m1, rmax_ref[pl.ds(i0m + j, 1), :])
                ridx_ref[pl.ds(i0m + j, 1), :] = jnp.where(
                    ms, r1, ridx_ref[pl.ds(i0m + j, 1), :])
        return carry

    jax.lax.fori_loop(0, _NSTEPS, step, 0)

    lt = (jax.lax.broadcasted_iota(jnp.int32, (_NSTEPS, _NSTEPS), 1)
          < jax.lax.broadcasted_iota(jnp.int32, (_NSTEPS, _NSTEPS), 0))
    accs, firsts, maccs = [], [], []
    for q in range(_NSTREAMS):
        colq = rit_ref[:, q:q + 1]
        row = ri_ref[q:q + 1, :]
        vrow = rv_ref[q:q + 1, :]
        eq = colq == row
        acc = jnp.sum(jnp.where(eq, vrow, 0.0), axis=1, keepdims=True)
        prior = jnp.sum(jnp.where(eq & lt, 1.0, 0.0), axis=1, keepdims=True)
        first = prior == 0.0
        accs.append(acc)
        firsts.append(first)
        maccs.append(jnp.max(jnp.where(first, acc, -jnp.inf)))
    mx = jnp.maximum(0.0, functools.reduce(jnp.maximum, maccs))

    total = 0.0
    for b in range(_BATCH):
        tb = b + _BATCH
        rcol = rit_ref[:, b:b + 1]
        t_row_i = ri_ref[tb:tb + 1, :]
        t_row_v = rv_ref[tb:tb + 1, :]
        eqc = rcol == t_row_i
        t_at_r = jnp.sum(jnp.where(eqc, t_row_v, 0.0), axis=1,
                         keepdims=True) / mx
        rn = accs[b] / mx
        ell = -(t_at_r * _clamped_log(rn)
                + (1.0 - t_at_r) * _clamped_log(1.0 - rn))
        total += jnp.sum(jnp.where(firsts[b], ell, 0.0))
        tcol = rit_ref[:, tb:tb + 1]
        r_row_i = ri_ref[b:b + 1, :]
        in_r = jnp.sum((tcol == r_row_i).astype(jnp.float32), axis=1,
                       keepdims=True) > 0.0
        tn = accs[tb] / mx
        total += jnp.sum(jnp.where(firsts[tb] & jnp.logical_not(in_r),
                                   100.0 * tn, 0.0))

    out_ref[...] = jnp.full((8, 128), total / (_BATCH * _NA * _N), jnp.float32)


def kernel(recon, target, d):
    sig = jnp.concatenate([recon.reshape(_BATCH, _N),
                           target.reshape(_BATCH, _N)], axis=0)
    blk = jnp.pad(sig, ((0, 0), (0, _AS))).reshape(_NSTREAMS, _NBLK + 1, _AS)
    x = jnp.concatenate([blk[:, :_NBLK, :], blk[:, 1:, :]],
                        axis=-1).reshape(_NSTREAMS * _NBLK, 2 * _AS)
    vall = jnp.pad(d, ((0, 0), (0, _AS + 1)))
    tiled = jnp.tile(vall, (1, _AS))[:, :_AS * 2 * _AS].reshape(_NA, _AS, 2 * _AS)
    m12 = jnp.transpose(tiled, (2, 1, 0)).reshape(2 * _AS, _C)

    out = pl.pallas_call(
        _mp_kernel,
        in_specs=[
            pl.BlockSpec((_NSTREAMS * _NBLK, 2 * _AS), lambda: (0, 0)),
            pl.BlockSpec((2 * _AS, _C), lambda: (0, 0)),
            pl.BlockSpec((_NA, _AS), lambda: (0, 0)),
        ],
        out_specs=pl.BlockSpec((8, 128), lambda: (0, 0)),
        out_shape=jax.ShapeDtypeStruct((8, 128), jnp.float32),
        scratch_shapes=[
            pltpu.VMEM((_NSTREAMS * _NBLK, _C), jnp.float32),
            pltpu.VMEM((2 * _NA, _C), jnp.float32),
            pltpu.VMEM((_NA, _AS), jnp.float32),
            pltpu.VMEM((_NBLK, _NSTREAMS), jnp.float32),
            pltpu.VMEM((_NBLK, _NSTREAMS), jnp.int32),
            pltpu.VMEM((_NSTEPS, _NSTEPS), jnp.float32),
            pltpu.VMEM((_NSTEPS, _NSTEPS), jnp.int32),
            pltpu.VMEM((_NSTEPS, _NSTEPS), jnp.int32),
        ],
    )(x, m12, d)
    return out[0, 0]
